# Initial kernel scaffold; baseline (speedup 1.0000x reference)
import jax
import jax.numpy as jnp
from jax.experimental import pallas as pl


def kernel(x, batch, W):
    raise NotImplementedError("write your pallas kernel here")


# speedup vs baseline: 6.0319x; 6.0319x over previous
import functools

import jax
import jax.numpy as jnp
from jax import lax
from jax.experimental import pallas as pl
from jax.experimental.pallas import tpu as pltpu
from jax.experimental.pallas import tpu_sc as plsc

N_GRAPHS = 256
CHUNK = 512
G_PER_W = N_GRAPHS // 32
DC = 8


def _wid():
    return lax.axis_index("s") * 2 + lax.axis_index("c")


def _make_pass1(n_nodes, dim):
    mesh = plsc.VectorSubcoreMesh(core_axis_name="c", subcore_axis_name="s")

    @functools.partial(
        pl.kernel,
        mesh=mesh,
        compiler_params=pltpu.CompilerParams(needs_layout_passes=False),
        out_type=jax.ShapeDtypeStruct((N_GRAPHS, dim), jnp.float32),
        scratch_types=[
            pltpu.VMEM((16,), jnp.int32),
            pltpu.VMEM((CHUNK, dim), jnp.float32),
            pltpu.VMEM((G_PER_W, dim), jnp.float32),
        ],
    )
    def pass1(x_hbm, starts_hbm, sums_hbm, sv, buf, acc):
        w = _wid()
        pltpu.sync_copy(starts_hbm.at[pl.ds(w * G_PER_W, 16)], sv)
        zero = jnp.zeros((16,), jnp.float32)
        for gi in range(G_PER_W):
            for c in range(DC):
                acc[gi, pl.ds(c * 16, 16)] = zero
        svv = sv[...]
        s_lo = svv[0]
        s_hi = svv[G_PER_W]
        base = (s_lo // 8) * 8
        nch = (s_hi - base + CHUNK - 1) // CHUNK

        def chunk_body(k, _):
            c0 = base + k * CHUNK
            off = pl.multiple_of(jnp.minimum(c0, n_nodes - CHUNK), 8)
            pltpu.sync_copy(x_hbm.at[pl.ds(off, CHUNK), :], buf)
            c1 = jnp.minimum(c0 + CHUNK, s_hi)
            for gi in range(G_PER_W):
                lo = jnp.maximum(svv[gi], c0)
                hi = jnp.minimum(svv[gi + 1], c1)

                @pl.when(hi > lo)
                def _():
                    init = tuple(acc[gi, pl.ds(c * 16, 16)] for c in range(DC))

                    def row(r, carry):
                        rl = r - off
                        return tuple(
                            carry[c] + buf[rl, pl.ds(c * 16, 16)]
                            for c in range(DC)
                        )

                    res = lax.fori_loop(lo, hi, row, init)
                    for c in range(DC):
                        acc[gi, pl.ds(c * 16, 16)] = res[c]
            return 0

        lax.fori_loop(0, nch, chunk_body, 0)
        pltpu.sync_copy(acc, sums_hbm.at[pl.ds(w * G_PER_W, G_PER_W), :])

    return pass1


def _make_pass2(n_nodes, dim):
    mesh = plsc.VectorSubcoreMesh(core_axis_name="c", subcore_axis_name="s")

    @functools.partial(
        pl.kernel,
        mesh=mesh,
        compiler_params=pltpu.CompilerParams(needs_layout_passes=False),
        out_type=jax.ShapeDtypeStruct((N_GRAPHS, dim), jnp.float32),
        scratch_types=[
            pltpu.VMEM((16,), jnp.int32),
            pltpu.VMEM((CHUNK, dim), jnp.float32),
            pltpu.VMEM((G_PER_W, dim), jnp.float32),
            pltpu.VMEM((G_PER_W, dim), jnp.float32),
        ],
    )
    def pass2(x_hbm, starts_hbm, tg_hbm, out_hbm, sv, buf, tgq, acc):
        w = _wid()
        pltpu.sync_copy(starts_hbm.at[pl.ds(w * G_PER_W, 16)], sv)
        pltpu.sync_copy(tg_hbm.at[pl.ds(w * G_PER_W, G_PER_W), :], tgq)
        zero = jnp.zeros((16,), jnp.float32)
        for gi in range(G_PER_W):
            for c in range(DC):
                acc[gi, pl.ds(c * 16, 16)] = zero
        svv = sv[...]
        s_lo = svv[0]
        s_hi = svv[G_PER_W]
        base = (s_lo // 8) * 8
        nch = (s_hi - base + CHUNK - 1) // CHUNK

        def chunk_body(k, _):
            c0 = base + k * CHUNK
            off = pl.multiple_of(jnp.minimum(c0, n_nodes - CHUNK), 8)
            pltpu.sync_copy(x_hbm.at[pl.ds(off, CHUNK), :], buf)
            c1 = jnp.minimum(c0 + CHUNK, s_hi)
            for gi in range(G_PER_W):
                lo = jnp.maximum(svv[gi], c0)
                hi = jnp.minimum(svv[gi + 1], c1)

                @pl.when(hi > lo)
                def _():
                    tgv = tuple(tgq[gi, pl.ds(c * 16, 16)] for c in range(DC))
                    init = tuple(acc[gi, pl.ds(c * 16, 16)] for c in range(DC))

                    def row(r, carry):
                        rl = r - off
                        xv = [buf[rl, pl.ds(c * 16, 16)] for c in range(DC)]
                        part = xv[0] * tgv[0]
                        for c in range(1, DC):
                            part = part + xv[c] * tgv[c]
                        s = jnp.sum(part) * 10.0
                        z = jnp.full((16,), s, jnp.float32)
                        coef = 1.0 / (1.0 + jnp.exp(-z))
                        return tuple(carry[c] + coef * xv[c] for c in range(DC))

                    res = lax.fori_loop(lo, hi, row, init)
                    for c in range(DC):
                        acc[gi, pl.ds(c * 16, 16)] = res[c]
            return 0

        lax.fori_loop(0, nch, chunk_body, 0)
        pltpu.sync_copy(acc, out_hbm.at[pl.ds(w * G_PER_W, G_PER_W), :])

    return pass2


def _mid_body(sums_ref, cnt_ref, w_ref, tg_ref):
    mean = sums_ref[...] / cnt_ref[...]
    tg_ref[...] = jnp.tanh(
        jnp.dot(mean, w_ref[...], preferred_element_type=jnp.float32)
    )


def kernel(x, batch, W):
    n_nodes, dim = x.shape
    batch = batch.astype(jnp.int32)
    starts = jnp.searchsorted(
        batch, jnp.arange(N_GRAPHS, dtype=jnp.int32)
    ).astype(jnp.int32)
    starts_ext = jnp.concatenate(
        [starts, jnp.full((16,), n_nodes, jnp.int32)]
    )
    counts = jnp.maximum(
        (starts_ext[1 : N_GRAPHS + 1] - starts).astype(jnp.float32), 1.0
    ).reshape(N_GRAPHS, 1)

    sums = _make_pass1(n_nodes, dim)(x, starts_ext)

    tg = pl.pallas_call(
        _mid_body,
        out_shape=jax.ShapeDtypeStruct((N_GRAPHS, dim), jnp.float32),
    )(sums, counts, W)

    return _make_pass2(n_nodes, dim)(x, starts_ext, tg)
